# baseline (speedup 1.0000x reference)
import functools

import jax
import jax.numpy as jnp
from jax import lax
from jax.experimental import pallas as pl
from jax.experimental.pallas import tpu as pltpu
from jax.experimental.pallas import tpu_sc as plsc

E = 8
K = 2
D = 768
H = 256
H2 = H // 2
N = 16384
EPS = 1e-5

BT = 2048
NW = 32
CHUNK = N // NW
NG = CHUNK // 16


def _normalize(x):
    m = jnp.mean(x, axis=-1, keepdims=True)
    v = jnp.mean(x * x, axis=-1, keepdims=True) - m * m
    return (x - m) * jax.lax.rsqrt(v + EPS)


def _gelu(x):
    return 0.5 * x * (1.0 + jax.lax.erf(x * (2.0 ** -0.5)))


def _rowstats(h):
    m = jnp.mean(h, axis=-1, keepdims=True)
    v = jnp.mean(h * h, axis=-1, keepdims=True) - m * m
    return m, jax.lax.rsqrt(v + EPS)


def _expert_block(
    x_ref, g_in_ref, b_in_ref, g_r_ref, b_r_ref, W_r_ref, br_ref,
    ln1gT_ref, ln1b_ref, W1_ref, b1_ref, ln2gT_ref, ln2b_ref,
    W2_ref, b2_ref, ln3g_ref, ln3gT_ref, ln3bT_ref, W3_ref, W3T_ref,
    b3_ref,
    outs_ref, logits_ref, logT_ref,
    Wr_s, cr_s, W1_s, c1_s, W2_s, s2_s, c2_s, w3_s, s3_s, c3_s,
):
    i = pl.program_id(0)

    @pl.when(i == 0)
    def _():
        Wr_s[...] = g_r_ref[...] * W_r_ref[...]
        cr_s[...] = (jnp.dot(b_r_ref[...], W_r_ref[...],
                             preferred_element_type=jnp.float32)
                     + br_ref[...])
        s3_s[...] = jnp.sum(ln3gT_ref[...] * W3T_ref[...],
                            axis=0, keepdims=True)
        c3_s[...] = (jnp.sum(ln3bT_ref[...] * W3T_ref[...],
                             axis=0, keepdims=True)
                     + b3_ref[...])
        for e in range(E):
            W1_s[e] = (ln1gT_ref[:, e:e + 1]
                       * W1_ref[e]).astype(jnp.bfloat16)
            c1_s[e:e + 1, :] = (
                jnp.dot(ln1b_ref[e:e + 1, :], W1_ref[e],
                        preferred_element_type=jnp.float32)
                + b1_ref[e:e + 1, :])
            w2f = ln2gT_ref[:, e:e + 1] * W2_ref[e]
            W2_s[e] = w2f.astype(jnp.bfloat16)
            s2_s[e:e + 1, :] = jnp.sum(w2f, axis=0, keepdims=True)
            c2_s[e:e + 1, :] = (
                jnp.dot(ln2b_ref[e:e + 1, :], W2_ref[e],
                        preferred_element_type=jnp.float32)
                + b2_ref[e:e + 1, :])
            w3_s[e:e + 1, :] = ln3g_ref[e:e + 1, :] * W3_ref[e:e + 1, :]

    x = x_ref[...]
    u = _normalize(x) * g_in_ref[...] + b_in_ref[...]
    z = _normalize(u)
    zb = z.astype(jnp.bfloat16)

    logits = (jnp.dot(z, Wr_s[...], preferred_element_type=jnp.float32)
              + cr_s[...])
    logits_ref[...] = logits
    logT_ref[...] = logits.T

    dots, mh3s, rh3s = [], [], []
    for e in range(E):
        h = _gelu(jnp.dot(zb, W1_s[e], preferred_element_type=jnp.float32)
                  + c1_s[e:e + 1, :])
        mh, rh = _rowstats(h)
        h = (jnp.dot(h.astype(jnp.bfloat16), W2_s[e],
                     preferred_element_type=jnp.float32)
             - mh * s2_s[e:e + 1, :]) * rh + c2_s[e:e + 1, :]
        h = _gelu(h)
        mh3, rh3 = _rowstats(h)
        dots.append(jnp.sum(h * w3_s[e:e + 1, :], axis=-1, keepdims=True))
        mh3s.append(mh3)
        rh3s.append(rh3)
    dot_c = jnp.concatenate(dots, axis=1)
    mh3_c = jnp.concatenate(mh3s, axis=1)
    rh3_c = jnp.concatenate(rh3s, axis=1)
    outs_ref[...] = rh3_c * (dot_c - mh3_c * s3_s[...]) + c3_s[...]


def _sc_router(logT_hbm, swT_hbm, fp_hbm, pp_hbm, lg_v, sw_v, f_v, p_v):
    wid = lax.axis_index("s") * 2 + lax.axis_index("c")
    base = wid * CHUNK
    for e in range(E):
        pltpu.sync_copy(logT_hbm.at[e, pl.ds(base, CHUNK)], lg_v.at[e])

    zf = jnp.zeros((16,), jnp.float32)
    zi = jnp.zeros((16,), jnp.int32)
    one = jnp.full((16,), 1.0, jnp.float32)
    ninf = jnp.full((16,), -jnp.inf, jnp.float32)
    evecs = [jnp.full((16,), e, jnp.int32) for e in range(E)]

    accf = [zf] * E
    accp = [zf] * E
    for g in range(NG):
        ls = [lg_v[e, pl.ds(g * 16, 16)] for e in range(E)]
        m1 = ls[0]
        i1 = zi
        for e in range(1, E):
            gt = ls[e] > m1
            m1 = jnp.where(gt, ls[e], m1)
            i1 = jnp.where(gt, evecs[e], i1)
        m2 = ninf
        i2 = zi
        for e in range(E):
            gt = (ls[e] > m2) & (i1 != evecs[e])
            m2 = jnp.where(gt, ls[e], m2)
            i2 = jnp.where(gt, evecs[e], i2)
        t = jnp.exp(m2 - m1)
        rden = one / (one + t)
        w1 = rden
        w2 = t * rden
        pes = [jnp.exp(ls[e] - m1) for e in range(E)]
        s = pes[0]
        for e in range(1, E):
            s = s + pes[e]
        rs = one / s
        w2pos = w2 > zf
        for e in range(E):
            is1 = i1 == evecs[e]
            is2 = i2 == evecs[e]
            swe = jnp.where(is1, w1, zf) + jnp.where(is2, w2, zf)
            sw_v[e, pl.ds(g * 16, 16)] = swe
            rt = is1 | (is2 & w2pos)
            accf[e] = accf[e] + jnp.where(rt, one, zf)
            accp[e] = accp[e] + pes[e] * rs
    for e in range(E):
        f_v[e, :] = accf[e]
        p_v[e, :] = accp[e]
    for e in range(E):
        pltpu.sync_copy(sw_v.at[e], swT_hbm.at[e, pl.ds(base, CHUNK)])
    pltpu.sync_copy(f_v, fp_hbm.at[wid])
    pltpu.sync_copy(p_v, pp_hbm.at[wid])


def _combine_block(outs_ref, sw_ref, fp_ref, pp_ref,
                   out_ref, frac_ref, prob_ref):
    i = pl.program_id(0)
    out_ref[...] = jnp.sum(outs_ref[...] * sw_ref[...],
                           axis=1, keepdims=True)

    @pl.when(i == 0)
    def _():
        frac_ref[...] = jnp.sum(jnp.sum(fp_ref[...], axis=2),
                                axis=0, keepdims=True) * (1.0 / N)
        prob_ref[...] = jnp.sum(jnp.sum(pp_ref[...], axis=2),
                                axis=0, keepdims=True) * (1.0 / N)


def kernel(x, ln_in_g, ln_in_b, ln_r_g, ln_r_b, W_r, b_r,
           e_ln1_g, e_ln1_b, e_W1, e_b1, e_ln2_g, e_ln2_b,
           e_W2, e_b2, e_ln3_g, e_ln3_b, e_W3, e_b3):
    nb = N // BT

    def rep(shape):
        return pl.BlockSpec(shape, lambda i: (0,) * len(shape))

    outs, logits, logT = pl.pallas_call(
        _expert_block,
        grid=(nb,),
        in_specs=[
            pl.BlockSpec((BT, D), lambda i: (i, 0)),
            rep((1, D)), rep((1, D)), rep((D, 1)), rep((1, D)),
            rep((D, E)), rep((1, E)),
            rep((D, E)), rep((E, D)), rep((E, D, H)), rep((E, H)),
            rep((H, E)), rep((E, H)), rep((E, H, H2)), rep((E, H2)),
            rep((E, H2)), rep((H2, E)), rep((H2, E)), rep((E, H2)),
            rep((H2, E)), rep((1, E)),
        ],
        out_specs=[
            pl.BlockSpec((BT, E), lambda i: (i, 0)),
            pl.BlockSpec((BT, E), lambda i: (i, 0)),
            pl.BlockSpec((E, BT), lambda i: (0, i)),
        ],
        out_shape=[
            jax.ShapeDtypeStruct((N, E), jnp.float32),
            jax.ShapeDtypeStruct((N, E), jnp.float32),
            jax.ShapeDtypeStruct((E, N), jnp.float32),
        ],
        scratch_shapes=[
            pltpu.VMEM((D, E), jnp.float32),
            pltpu.VMEM((1, E), jnp.float32),
            pltpu.VMEM((E, D, H), jnp.bfloat16),
            pltpu.VMEM((E, H), jnp.float32),
            pltpu.VMEM((E, H, H2), jnp.bfloat16),
            pltpu.VMEM((E, H2), jnp.float32),
            pltpu.VMEM((E, H2), jnp.float32),
            pltpu.VMEM((E, H2), jnp.float32),
            pltpu.VMEM((1, E), jnp.float32),
            pltpu.VMEM((1, E), jnp.float32),
        ],
        compiler_params=pltpu.CompilerParams(
            dimension_semantics=("arbitrary",),
        ),
    )(
        x,
        ln_in_g.reshape(1, D), ln_in_b.reshape(1, D),
        ln_r_g.reshape(D, 1), ln_r_b.reshape(1, D),
        W_r, b_r.reshape(1, E),
        e_ln1_g.T, e_ln1_b, e_W1, e_b1,
        e_ln2_g.T, e_ln2_b, e_W2, e_b2,
        e_ln3_g, e_ln3_g.T, e_ln3_b.T,
        e_W3.reshape(E, H2), e_W3.reshape(E, H2).T, e_b3.reshape(1, E),
    )

    mesh = plsc.VectorSubcoreMesh(core_axis_name="c", subcore_axis_name="s")
    swT, fp, pp = pl.kernel(
        _sc_router,
        mesh=mesh,
        out_type=[
            jax.ShapeDtypeStruct((E, N), jnp.float32),
            jax.ShapeDtypeStruct((NW, E, 16), jnp.float32),
            jax.ShapeDtypeStruct((NW, E, 16), jnp.float32),
        ],
        scratch_types=[
            pltpu.VMEM((E, CHUNK), jnp.float32),
            pltpu.VMEM((E, CHUNK), jnp.float32),
            pltpu.VMEM((E, 16), jnp.float32),
            pltpu.VMEM((E, 16), jnp.float32),
        ],
    )(logT)

    out, frac16, prob16 = pl.pallas_call(
        _combine_block,
        grid=(nb,),
        in_specs=[
            pl.BlockSpec((BT, E), lambda i: (i, 0)),
            pl.BlockSpec((BT, E), lambda i: (i, 0)),
            rep((NW, E, 16)), rep((NW, E, 16)),
        ],
        out_specs=[
            pl.BlockSpec((BT, 1), lambda i: (i, 0)),
            pl.BlockSpec((1, E), lambda i: (0, 0)),
            pl.BlockSpec((1, E), lambda i: (0, 0)),
        ],
        out_shape=[
            jax.ShapeDtypeStruct((N, 1), jnp.float32),
            jax.ShapeDtypeStruct((1, E), jnp.float32),
            jax.ShapeDtypeStruct((1, E), jnp.float32),
        ],
        compiler_params=pltpu.CompilerParams(
            dimension_semantics=("arbitrary",),
        ),
    )(outs, swT.T, fp, pp)

    return (out, frac16.reshape(E), prob16.reshape(E), logits)
